# baseline (speedup 1.0000x reference)
import jax
import jax.numpy as jnp
from jax.experimental import pallas as pl
from jax.experimental.pallas import tpu as pltpu

N = 10000
D = 128
BM = 400


def _gcn_body(adj_ref, emb_hbm_ref, out_ref, emb_f32_ref, emb_bf_ref, sem):
    @pl.when(pl.program_id(0) == 0)
    def _():
        copy = pltpu.make_async_copy(emb_hbm_ref, emb_f32_ref, sem)
        copy.start()
        copy.wait()
        emb_bf_ref[...] = emb_f32_ref[...].astype(jnp.bfloat16)

    a = adj_ref[...].astype(jnp.bfloat16)
    out_ref[...] = jnp.dot(a, emb_bf_ref[...], preferred_element_type=jnp.float32)


def kernel(adj, embeds):
    grid = (N // BM,)
    return pl.pallas_call(
        _gcn_body,
        grid=grid,
        in_specs=[
            pl.BlockSpec((BM, N), lambda i: (i, 0)),
            pl.BlockSpec(memory_space=pl.ANY),
        ],
        out_specs=pl.BlockSpec((BM, D), lambda i: (i, 0)),
        out_shape=jax.ShapeDtypeStruct((N, D), jnp.float32),
        scratch_shapes=[
            pltpu.VMEM((N, D), jnp.float32),
            pltpu.VMEM((N, D), jnp.bfloat16),
            pltpu.SemaphoreType.DMA,
        ],
        compiler_params=pltpu.CompilerParams(
            dimension_semantics=("arbitrary",),
        ),
    )(adj, embeds)


# speedup vs baseline: 1.0230x; 1.0230x over previous
import jax
import jax.numpy as jnp
from jax.experimental import pallas as pl
from jax.experimental.pallas import tpu as pltpu

N = 10000
D = 128
BM = 400


def _gcn_body(adj_ref, emb_ref, out_ref, emb_bf_ref):
    i = pl.program_id(0)

    @pl.when(i == 0)
    def _():
        emb_bf_ref[...] = emb_ref[...].astype(jnp.bfloat16)

    a = adj_ref[...].astype(jnp.bfloat16)
    out_ref[pl.ds(i * BM, BM), :] = jnp.dot(
        a, emb_bf_ref[...], preferred_element_type=jnp.float32)


def kernel(adj, embeds):
    grid = (N // BM,)
    return pl.pallas_call(
        _gcn_body,
        grid=grid,
        in_specs=[
            pl.BlockSpec((BM, N), lambda i: (i, 0)),
            pl.BlockSpec((N, D), lambda i: (0, 0)),
        ],
        out_specs=pl.BlockSpec((N, D), lambda i: (0, 0)),
        out_shape=jax.ShapeDtypeStruct((N, D), jnp.float32),
        scratch_shapes=[pltpu.VMEM((N, D), jnp.bfloat16)],
        compiler_params=pltpu.CompilerParams(
            dimension_semantics=("arbitrary",),
        ),
    )(adj, embeds)


# speedup vs baseline: 1.0272x; 1.0041x over previous
import jax
import jax.numpy as jnp
from jax.experimental import pallas as pl
from jax.experimental.pallas import tpu as pltpu

N = 10000
D = 128
BM = 400


def _gcn_body(adj_ref, emb_ref, out_ref, emb_bf_ref):
    @pl.when(pl.program_id(0) == 0)
    def _():
        emb_bf_ref[...] = emb_ref[...].astype(jnp.bfloat16)

    a = adj_ref[...].astype(jnp.bfloat16)
    out_ref[...] = jnp.dot(a, emb_bf_ref[...], preferred_element_type=jnp.float32)


def kernel(adj, embeds):
    grid = (pl.cdiv(N, BM),)
    return pl.pallas_call(
        _gcn_body,
        grid=grid,
        in_specs=[
            pl.BlockSpec((BM, N), lambda i: (i, 0)),
            pl.BlockSpec((N, D), lambda i: (0, 0)),
        ],
        out_specs=pl.BlockSpec((BM, D), lambda i: (i, 0)),
        out_shape=jax.ShapeDtypeStruct((N, D), jnp.float32),
        scratch_shapes=[pltpu.VMEM((N, D), jnp.bfloat16)],
        compiler_params=pltpu.CompilerParams(
            dimension_semantics=("arbitrary",),
        ),
    )(adj, embeds)


# speedup vs baseline: 1.0309x; 1.0036x over previous
import jax
import jax.numpy as jnp
from jax.experimental import pallas as pl
from jax.experimental.pallas import tpu as pltpu

N = 10000
D = 128
BC = 200
NC = N // BC
NBUF = 3
NOBUF = 2


def _adj_copy(adj_hbm, adj_buf, sems, chunk, slot):
    return pltpu.make_async_copy(
        adj_hbm.at[pl.ds(chunk * BC, BC), :], adj_buf.at[slot], sems.at[slot])


def _out_copy(out_buf, out_hbm, sems, chunk, slot):
    return pltpu.make_async_copy(
        out_buf.at[slot], out_hbm.at[pl.ds(chunk * BC, BC), :], sems.at[slot])


def _body(emb_ref, adj_hbm, out_hbm, emb_bf, adj_buf, out_buf, adj_sems, out_sems):
    emb_bf[...] = emb_ref[...].astype(jnp.bfloat16)

    for s in range(NBUF):
        _adj_copy(adj_hbm, adj_buf, adj_sems, s, s).start()

    def step(i, _):
        slot = jax.lax.rem(i, NBUF)
        oslot = jax.lax.rem(i, NOBUF)
        _adj_copy(adj_hbm, adj_buf, adj_sems, i, slot).wait()
        a = adj_buf[slot].astype(jnp.bfloat16)
        o = jnp.dot(a, emb_bf[...], preferred_element_type=jnp.float32)

        @pl.when(i >= NOBUF)
        def _():
            _out_copy(out_buf, out_hbm, out_sems, i - NOBUF, oslot).wait()

        out_buf[oslot] = o
        _out_copy(out_buf, out_hbm, out_sems, i, oslot).start()

        @pl.when(i + NBUF < NC)
        def _():
            _adj_copy(adj_hbm, adj_buf, adj_sems, i + NBUF,
                      jax.lax.rem(i + NBUF, NBUF)).start()
        return 0

    jax.lax.fori_loop(0, NC, step, 0)

    for t in range(NOBUF):
        c = NC - NOBUF + t
        _out_copy(out_buf, out_hbm, out_sems, c, c % NOBUF).wait()


def kernel(adj, embeds):
    return pl.pallas_call(
        _body,
        grid=(1,),
        in_specs=[
            pl.BlockSpec((N, D), lambda i: (0, 0)),
            pl.BlockSpec(memory_space=pl.ANY),
        ],
        out_specs=pl.BlockSpec(memory_space=pl.ANY),
        out_shape=jax.ShapeDtypeStruct((N, D), jnp.float32),
        scratch_shapes=[
            pltpu.VMEM((N, D), jnp.bfloat16),
            pltpu.VMEM((NBUF, BC, N), jnp.float32),
            pltpu.VMEM((NOBUF, BC, D), jnp.float32),
            pltpu.SemaphoreType.DMA((NBUF,)),
            pltpu.SemaphoreType.DMA((NOBUF,)),
        ],
        compiler_params=pltpu.CompilerParams(
            dimension_semantics=("arbitrary",),
        ),
    )(embeds, adj)
